# baseline (speedup 1.0000x reference)
import jax
import jax.numpy as jnp
from jax import lax
from jax.experimental import pallas as pl
from jax.experimental.pallas import tpu as pltpu
from jax.experimental.pallas import tpu_sc as plsc

N = 10000
E = 160000
F_IN = 128
H = 4
F_OUT = 32
HF = H * F_OUT
B = 4
T = 8
NI = B * T
ALPHA = 0.2

BN = 1000
NB = N // BN

K = 128
NCHUNKS = E // K
NSUB = 16
NPT = N // NSUB
NROW = 125
NQ = NPT // NROW


def _tc_body(x_ref, w_ref, asr_ref, adr_ref, h_ref, esed_ref):
    xb = x_ref[0, 0]
    h = jnp.dot(xb, w_ref[...], preferred_element_type=jnp.float32)
    h_ref[...] = h
    fi = lax.broadcasted_iota(jnp.int32, (HF, H), 0) // F_OUT
    hi = lax.broadcasted_iota(jnp.int32, (HF, H), 1)
    sel = (fi == hi).astype(jnp.float32)
    es = jnp.dot(h * asr_ref[...], sel, preferred_element_type=jnp.float32)
    ed = jnp.dot(h * adr_ref[...], sel, preferred_element_type=jnp.float32)
    pad = jnp.zeros((BN, 8), jnp.float32)
    esed_ref[...] = jnp.concatenate([es, ed, pad], axis=1)


def _tc_stage(x, W, a_src2, a_dst2):
    return pl.pallas_call(
        _tc_body,
        grid=(NI, NB),
        in_specs=[
            pl.BlockSpec((1, 1, BN, F_IN), lambda i, nb: (i % B, i // B, nb, 0)),
            pl.BlockSpec((F_IN, HF), lambda i, nb: (0, 0)),
            pl.BlockSpec((1, HF), lambda i, nb: (0, 0)),
            pl.BlockSpec((1, HF), lambda i, nb: (0, 0)),
        ],
        out_specs=[
            pl.BlockSpec((BN, HF), lambda i, nb: (i * NB + nb, 0)),
            pl.BlockSpec((BN, 16), lambda i, nb: (i * NB + nb, 0)),
        ],
        out_shape=[
            jax.ShapeDtypeStruct((NI * N, HF), jnp.float32),
            jax.ShapeDtypeStruct((NI * N, 16), jnp.float32),
        ],
    )(x, W, a_src2, a_dst2)


def _sc_body(h_hbm, esed_hbm, src_hbm, dst_hbm, num_hbm, den_hbm,
             num_sp, den_sp,
             srcb, dstb, adjb, adjb2, hb, esb, edb, wb, zdenb, sem, sem2):
    cid = lax.axis_index("c")
    sid = lax.axis_index("s")
    lane = lax.iota(jnp.int32, 16)
    zero16 = jnp.zeros((16,), jnp.float32)

    def _zero_hb(k, _):
        for c8 in range(HF // 16):
            hb[k, pl.ds(c8 * 16, 16)] = zero16
        return 0

    def _zero_16wide(ref):
        def body(k, _):
            ref[k, pl.ds(0, 16)] = zero16
            return 0
        lax.fori_loop(0, K, body, 0)

    _zero_16wide(wb)
    _zero_16wide(zdenb)
    lax.fori_loop(0, K, _zero_hb, 0)

    for q in range(NQ):
        row0 = sid * NPT + q * NROW
        pltpu.sync_copy(hb.at[pl.ds(0, NROW)], num_sp.at[pl.ds(row0, NROW)])
        pltpu.sync_copy(zdenb.at[pl.ds(0, NROW)],
                        den_sp.at[pl.ds(row0, NROW)])

    nch = (NCHUNKS - sid + NSUB - 1) // NSUB

    def inst_body(il, _):
        inst = cid * (NI // 2) + il
        base_row = inst * N
        plsc.subcore_barrier()

        def chunk_body(j, _):
            c = sid + j * NSUB
            ebase = c * K
            pltpu.sync_copy(src_hbm.at[pl.ds(ebase, K)], srcb)
            pltpu.sync_copy(dst_hbm.at[pl.ds(ebase, K)], dstb)

            def _adj(i, _):
                adjb[pl.ds(i * 16, 16)] = srcb[pl.ds(i * 16, 16)] + base_row
                adjb2[pl.ds(i * 16, 16)] = dstb[pl.ds(i * 16, 16)] + base_row
                return 0
            lax.fori_loop(0, K // 16, _adj, 0, unroll=4)
            hcopy = pltpu.async_copy(h_hbm.at[adjb], hb, sem)
            e1copy = pltpu.async_copy(esed_hbm.at[adjb], esb, sem2)
            e2copy = pltpu.async_copy(esed_hbm.at[adjb2], edb, sem2)
            e1copy.wait()
            e2copy.wait()

            def _wloop(i, _):
                jv = i * 16 + lane
                kv = jv >> 2
                hv = jv & 3
                e1 = plsc.load_gather(esb, [kv, hv])
                e2 = plsc.load_gather(edb, [kv, hv + 4])
                e = e1 + e2
                e = jnp.where(e > 0, e, ALPHA * e)
                plsc.store_scatter(wb, [kv, hv], jnp.exp(e))
                return 0
            lax.fori_loop(0, (K * H) // 16, _wloop, 0, unroll=4)
            hcopy.wait()

            def _sloop(k, _):
                for hd in range(H):
                    wv = plsc.load_gather(
                        wb,
                        [jnp.full((16,), k, jnp.int32),
                         jnp.full((16,), hd, jnp.int32)])
                    for half in range(2):
                        c8 = hd * 2 + half
                        hv = hb[k, pl.ds(c8 * 16, 16)]
                        hb[k, pl.ds(c8 * 16, 16)] = hv * wv
                return 0
            lax.fori_loop(0, K, _sloop, 0, unroll=2)

            pltpu.sync_copy(hb, num_sp.at[dstb], add=True)
            pltpu.sync_copy(wb, den_sp.at[dstb], add=True)
            return 0
        lax.fori_loop(0, nch, chunk_body, 0)
        plsc.subcore_barrier()

        nbase = sid * NPT
        pltpu.sync_copy(num_sp.at[pl.ds(nbase, NPT)],
                        num_hbm.at[pl.ds(base_row + nbase, NPT)])
        pltpu.sync_copy(den_sp.at[pl.ds(nbase, NPT)],
                        den_hbm.at[pl.ds(base_row + nbase, NPT)])
        lax.fori_loop(0, K, _zero_hb, 0)
        for q in range(NQ):
            row0 = nbase + q * NROW
            pltpu.sync_copy(hb.at[pl.ds(0, NROW)],
                            num_sp.at[pl.ds(row0, NROW)])
            pltpu.sync_copy(zdenb.at[pl.ds(0, NROW)],
                            den_sp.at[pl.ds(row0, NROW)])
        return 0
    lax.fori_loop(0, NI // 2, inst_body, 0)


def _sc_stage(h2, esed2, src, dst):
    mesh = plsc.VectorSubcoreMesh(core_axis_name="c", subcore_axis_name="s")
    return pl.kernel(
        _sc_body,
        out_type=[
            jax.ShapeDtypeStruct((NI * N, HF), jnp.float32),
            jax.ShapeDtypeStruct((NI * N, 16), jnp.float32),
        ],
        mesh=mesh,
        scratch_types=[
            pltpu.VMEM_SHARED((N, HF), jnp.float32),
            pltpu.VMEM_SHARED((N, 16), jnp.float32),
            pltpu.VMEM((K,), jnp.int32),
            pltpu.VMEM((K,), jnp.int32),
            pltpu.VMEM((K,), jnp.int32),
            pltpu.VMEM((K,), jnp.int32),
            pltpu.VMEM((K, HF), jnp.float32),
            pltpu.VMEM((K, 16), jnp.float32),
            pltpu.VMEM((K, 16), jnp.float32),
            pltpu.VMEM((K, 16), jnp.float32),
            pltpu.VMEM((K, 16), jnp.float32),
            pltpu.SemaphoreType.DMA,
            pltpu.SemaphoreType.DMA,
        ],
        compiler_params=pltpu.CompilerParams(
            use_tc_tiling_on_sc=False, needs_layout_passes=False),
    )(h2, esed2, src, dst)


def _fin_body(num_ref, den_ref, out_ref):
    ci = lax.broadcasted_iota(jnp.int32, (16, HF), 0)
    fi = lax.broadcasted_iota(jnp.int32, (16, HF), 1) // F_OUT
    sel2 = (ci == fi).astype(jnp.float32)
    den_exp = jnp.dot(den_ref[...], sel2, preferred_element_type=jnp.float32)
    v = num_ref[...] / (den_exp + 1e-16)
    out_ref[...] = jnp.where(v > 0, v, jnp.exp(jnp.minimum(v, 0.0)) - 1.0)


def _fin_stage(num2, den2):
    return pl.pallas_call(
        _fin_body,
        grid=(NI * NB,),
        in_specs=[
            pl.BlockSpec((BN, HF), lambda i: (i, 0)),
            pl.BlockSpec((BN, 16), lambda i: (i, 0)),
        ],
        out_specs=pl.BlockSpec((BN, HF), lambda i: (i, 0)),
        out_shape=jax.ShapeDtypeStruct((NI * N, HF), jnp.float32),
    )(num2, den2)


def kernel(x, edge_index, W, a_src, a_dst):
    a_src2 = a_src.reshape(1, HF)
    a_dst2 = a_dst.reshape(1, HF)
    h2, esed2 = _tc_stage(x, W, a_src2, a_dst2)
    src = edge_index[0]
    dst = edge_index[1]
    num2, den2 = _sc_stage(h2, esed2, src, dst)
    out2 = _fin_stage(num2, den2)
    return out2.reshape(T, B, N, HF)


# speedup vs baseline: 1.0748x; 1.0748x over previous
import jax
import jax.numpy as jnp
from jax import lax
from jax.experimental import pallas as pl
from jax.experimental.pallas import tpu as pltpu
from jax.experimental.pallas import tpu_sc as plsc

N = 10000
E = 160000
F_IN = 128
H = 4
F_OUT = 32
HF = H * F_OUT
B = 4
T = 8
NI = B * T
ALPHA = 0.2

BN = 1000
NB = N // BN

K = 128
NCHUNKS = E // K
NSUB = 16
NPT = N // NSUB
NROW = 125
NQ = NPT // NROW


def _tc_body(x_ref, w_ref, asr_ref, adr_ref, h_ref, esed_ref):
    xb = x_ref[0, 0]
    h = jnp.dot(xb, w_ref[...], preferred_element_type=jnp.float32)
    h_ref[...] = h
    fi = lax.broadcasted_iota(jnp.int32, (HF, H), 0) // F_OUT
    hi = lax.broadcasted_iota(jnp.int32, (HF, H), 1)
    sel = (fi == hi).astype(jnp.float32)
    es = jnp.dot(h * asr_ref[...], sel, preferred_element_type=jnp.float32)
    ed = jnp.dot(h * adr_ref[...], sel, preferred_element_type=jnp.float32)
    pad = jnp.zeros((BN, 8), jnp.float32)
    esed_ref[...] = jnp.concatenate([es, ed, pad], axis=1)


def _tc_stage(x, W, a_src2, a_dst2):
    return pl.pallas_call(
        _tc_body,
        grid=(NI, NB),
        in_specs=[
            pl.BlockSpec((1, 1, BN, F_IN), lambda i, nb: (i % B, i // B, nb, 0)),
            pl.BlockSpec((F_IN, HF), lambda i, nb: (0, 0)),
            pl.BlockSpec((1, HF), lambda i, nb: (0, 0)),
            pl.BlockSpec((1, HF), lambda i, nb: (0, 0)),
        ],
        out_specs=[
            pl.BlockSpec((BN, HF), lambda i, nb: (i * NB + nb, 0)),
            pl.BlockSpec((BN, 16), lambda i, nb: (i * NB + nb, 0)),
        ],
        out_shape=[
            jax.ShapeDtypeStruct((NI * N, HF), jnp.float32),
            jax.ShapeDtypeStruct((NI * N, 16), jnp.float32),
        ],
    )(x, W, a_src2, a_dst2)


def _sc_body(h_hbm, esed_hbm, src_hbm, dst_hbm, num_hbm, den_hbm,
             num_sp, den_sp,
             srcb, dstb, adjb, adjb2, hb, esb, edb, wb, zdenb,
             sem, sem2, sem3):
    cid = lax.axis_index("c")
    sid = lax.axis_index("s")
    lane = lax.iota(jnp.int32, 16)
    zero16 = jnp.zeros((16,), jnp.float32)

    def _zero_hb(k, _):
        for c8 in range(HF // 16):
            hb[k, pl.ds(c8 * 16, 16)] = zero16
        return 0

    def _zero_16wide(ref):
        def body(k, _):
            ref[k, pl.ds(0, 16)] = zero16
            return 0
        lax.fori_loop(0, K, body, 0)

    _zero_16wide(wb)
    _zero_16wide(zdenb)
    lax.fori_loop(0, K, _zero_hb, 0)

    for q in range(NQ):
        row0 = sid * NPT + q * NROW
        pltpu.sync_copy(hb.at[pl.ds(0, NROW)], num_sp.at[pl.ds(row0, NROW)])
        pltpu.sync_copy(zdenb.at[pl.ds(0, NROW)],
                        den_sp.at[pl.ds(row0, NROW)])

    nch = (NCHUNKS - sid + NSUB - 1) // NSUB

    def inst_body(il, _):
        inst = cid * (NI // 2) + il
        base_row = inst * N
        plsc.subcore_barrier()

        def _zidx(i, _):
            dstb[pl.ds(i * 16, 16)] = jnp.zeros((16,), jnp.int32)
            return 0
        lax.fori_loop(0, K // 16, _zidx, 0)
        pltpu.async_copy(hb, num_sp.at[dstb], sem3, add=True)
        pltpu.async_copy(zdenb, den_sp.at[dstb], sem3, add=True)

        def chunk_body(j, _):
            c = sid + j * NSUB
            ebase = c * K
            pltpu.sync_copy(src_hbm.at[pl.ds(ebase, K)], srcb)
            pltpu.make_async_copy(hb, num_sp.at[dstb], sem3).wait()
            pltpu.make_async_copy(wb, den_sp.at[dstb], sem3).wait()
            pltpu.sync_copy(dst_hbm.at[pl.ds(ebase, K)], dstb)

            def _adj(i, _):
                adjb[pl.ds(i * 16, 16)] = srcb[pl.ds(i * 16, 16)] + base_row
                adjb2[pl.ds(i * 16, 16)] = dstb[pl.ds(i * 16, 16)] + base_row
                return 0
            lax.fori_loop(0, K // 16, _adj, 0, unroll=4)
            hcopy = pltpu.async_copy(h_hbm.at[adjb], hb, sem)
            e1copy = pltpu.async_copy(esed_hbm.at[adjb], esb, sem2)
            e2copy = pltpu.async_copy(esed_hbm.at[adjb2], edb, sem2)
            e1copy.wait()
            e2copy.wait()

            def _wloop(i, _):
                jv = i * 16 + lane
                kv = jv >> 2
                hv = jv & 3
                e1 = plsc.load_gather(esb, [kv, hv])
                e2 = plsc.load_gather(edb, [kv, hv + 4])
                e = e1 + e2
                e = jnp.where(e > 0, e, ALPHA * e)
                plsc.store_scatter(wb, [kv, hv], jnp.exp(e))
                return 0
            lax.fori_loop(0, (K * H) // 16, _wloop, 0, unroll=4)
            hcopy.wait()

            def _sloop(k, _):
                for hd in range(H):
                    wv = plsc.load_gather(
                        wb,
                        [jnp.full((16,), k, jnp.int32),
                         jnp.full((16,), hd, jnp.int32)])
                    for half in range(2):
                        c8 = hd * 2 + half
                        hv = hb[k, pl.ds(c8 * 16, 16)]
                        hb[k, pl.ds(c8 * 16, 16)] = hv * wv
                return 0
            lax.fori_loop(0, K, _sloop, 0, unroll=2)

            pltpu.async_copy(hb, num_sp.at[dstb], sem3, add=True)
            pltpu.async_copy(wb, den_sp.at[dstb], sem3, add=True)
            return 0
        lax.fori_loop(0, nch, chunk_body, 0)
        pltpu.make_async_copy(hb, num_sp.at[dstb], sem3).wait()
        pltpu.make_async_copy(wb, den_sp.at[dstb], sem3).wait()
        plsc.subcore_barrier()

        nbase = sid * NPT
        pltpu.sync_copy(num_sp.at[pl.ds(nbase, NPT)],
                        num_hbm.at[pl.ds(base_row + nbase, NPT)])
        pltpu.sync_copy(den_sp.at[pl.ds(nbase, NPT)],
                        den_hbm.at[pl.ds(base_row + nbase, NPT)])
        lax.fori_loop(0, K, _zero_hb, 0)
        for q in range(NQ):
            row0 = nbase + q * NROW
            pltpu.sync_copy(hb.at[pl.ds(0, NROW)],
                            num_sp.at[pl.ds(row0, NROW)])
            pltpu.sync_copy(zdenb.at[pl.ds(0, NROW)],
                            den_sp.at[pl.ds(row0, NROW)])
        return 0
    lax.fori_loop(0, NI // 2, inst_body, 0)


def _sc_stage(h2, esed2, src, dst):
    mesh = plsc.VectorSubcoreMesh(core_axis_name="c", subcore_axis_name="s")
    return pl.kernel(
        _sc_body,
        out_type=[
            jax.ShapeDtypeStruct((NI * N, HF), jnp.float32),
            jax.ShapeDtypeStruct((NI * N, 16), jnp.float32),
        ],
        mesh=mesh,
        scratch_types=[
            pltpu.VMEM_SHARED((N, HF), jnp.float32),
            pltpu.VMEM_SHARED((N, 16), jnp.float32),
            pltpu.VMEM((K,), jnp.int32),
            pltpu.VMEM((K,), jnp.int32),
            pltpu.VMEM((K,), jnp.int32),
            pltpu.VMEM((K,), jnp.int32),
            pltpu.VMEM((K, HF), jnp.float32),
            pltpu.VMEM((K, 16), jnp.float32),
            pltpu.VMEM((K, 16), jnp.float32),
            pltpu.VMEM((K, 16), jnp.float32),
            pltpu.VMEM((K, 16), jnp.float32),
            pltpu.SemaphoreType.DMA,
            pltpu.SemaphoreType.DMA,
            pltpu.SemaphoreType.DMA,
        ],
        compiler_params=pltpu.CompilerParams(
            use_tc_tiling_on_sc=False, needs_layout_passes=False),
    )(h2, esed2, src, dst)


def _fin_body(num_ref, den_ref, out_ref):
    ci = lax.broadcasted_iota(jnp.int32, (16, HF), 0)
    fi = lax.broadcasted_iota(jnp.int32, (16, HF), 1) // F_OUT
    sel2 = (ci == fi).astype(jnp.float32)
    den_exp = jnp.dot(den_ref[...], sel2, preferred_element_type=jnp.float32)
    v = num_ref[...] / (den_exp + 1e-16)
    out_ref[...] = jnp.where(v > 0, v, jnp.exp(jnp.minimum(v, 0.0)) - 1.0)


def _fin_stage(num2, den2):
    return pl.pallas_call(
        _fin_body,
        grid=(NI * NB,),
        in_specs=[
            pl.BlockSpec((BN, HF), lambda i: (i, 0)),
            pl.BlockSpec((BN, 16), lambda i: (i, 0)),
        ],
        out_specs=pl.BlockSpec((BN, HF), lambda i: (i, 0)),
        out_shape=jax.ShapeDtypeStruct((NI * N, HF), jnp.float32),
    )(num2, den2)


def kernel(x, edge_index, W, a_src, a_dst):
    a_src2 = a_src.reshape(1, HF)
    a_dst2 = a_dst.reshape(1, HF)
    h2, esed2 = _tc_stage(x, W, a_src2, a_dst2)
    src = edge_index[0]
    dst = edge_index[1]
    num2, den2 = _sc_stage(h2, esed2, src, dst)
    out2 = _fin_stage(num2, den2)
    return out2.reshape(T, B, N, HF)
